# baseline (speedup 1.0000x reference)
import jax
import jax.numpy as jnp
from jax.experimental import pallas as pl
from jax.experimental.pallas import tpu as pltpu

_BM = 512


def _fused_kernel(b_ref, w_ref, bias_ref, at_ref, ab_ref, o0_ref, o1_ref, h_ref):
    @pl.when(pl.program_id(0) == 0)
    def _():
        h_ref[...] = (
            jnp.dot(b_ref[...], w_ref[...], preferred_element_type=jnp.float32)
            + bias_ref[...]
        )

    o0_ref[...] = jax.lax.dot_general(
        h_ref[...],
        at_ref[...],
        dimension_numbers=(((0,), (1,)), ((), ())),
        preferred_element_type=jnp.float32,
    )
    o1_ref[...] = jax.lax.dot_general(
        h_ref[...],
        ab_ref[...],
        dimension_numbers=(((0,), (1,)), ((), ())),
        preferred_element_type=jnp.float32,
    )


def kernel(a, b, W, bias):
    n, k = a.shape
    d_in = b.shape[1]
    d_out = W.shape[1]
    bias2d = bias.reshape(1, d_out)
    hb = n // (2 * _BM)

    o0, o1 = pl.pallas_call(
        _fused_kernel,
        grid=(hb,),
        in_specs=[
            pl.BlockSpec((k, d_in), lambda i: (0, 0)),
            pl.BlockSpec((d_in, d_out), lambda i: (0, 0)),
            pl.BlockSpec((1, d_out), lambda i: (0, 0)),
            pl.BlockSpec((_BM, k), lambda i: (i, 0)),
            pl.BlockSpec((_BM, k), lambda i: (i + hb, 0)),
        ],
        out_specs=[
            pl.BlockSpec((d_out, _BM), lambda i: (0, i)),
            pl.BlockSpec((d_out, _BM), lambda i: (0, i)),
        ],
        out_shape=[
            jax.ShapeDtypeStruct((d_out, n // 2), jnp.float32),
            jax.ShapeDtypeStruct((d_out, n // 2), jnp.float32),
        ],
        scratch_shapes=[pltpu.VMEM((k, d_out), jnp.float32)],
        compiler_params=pltpu.CompilerParams(
            dimension_semantics=("arbitrary",),
        ),
    )(b, W, bias2d, a, a)
    return jnp.concatenate([o0, o1], axis=1).T
